# baseline (speedup 1.0000x reference)
import functools
import numpy as np
import jax
import jax.numpy as jnp
from jax import lax
from jax.experimental import pallas as pl
from jax.experimental.pallas import tpu as pltpu
from jax.experimental.pallas import tpu_sc as plsc

_INTERPRET = False

M2 = 5120
B = 512
NB = M2 // B
NW = 32


def _pre_body(ct_ref, o_ref, st_ref, co_ref, sc_ref, lb_ref, ky_ref):
    x = ct_ref[0, 0:1, :]
    y = ct_ref[0, 1:2, :]
    w = ct_ref[0, 2:3, :]
    h = ct_ref[0, 3:4, :]
    one = jnp.float32(1.0)
    zero = jnp.float32(0.0)
    x1 = jnp.clip(x, zero, one)
    y1 = jnp.clip(y, zero, one)
    x2 = jnp.clip(x + w, zero, one)
    y2 = jnp.clip(y + h, zero, one)
    co_ref[0, 0:1, :] = x1
    co_ref[0, 1:2, :] = y1
    co_ref[0, 2:3, :] = x2
    co_ref[0, 3:4, :] = y2
    C = st_ref.shape[1]
    best = st_ref[0, 0:1, :]
    bi = jnp.zeros(best.shape, jnp.int32)
    for c in range(1, C):
        s = st_ref[0, c : c + 1, :]
        gt = s > best
        best = jnp.where(gt, s, best)
        bi = jnp.where(gt, jnp.int32(c), bi)
    sc_ref[0] = best
    lb_ref[0] = bi
    valid = o_ref[0] >= jnp.float32(0.5)
    ky_ref[0] = jnp.where(valid, -best, jnp.float32(jnp.inf))


def _iou_mat(x1c, y1c, x2c, y2c, ac, x1r, y1r, x2r, y2r, ar):
    xx1 = jnp.maximum(x1c, x1r)
    yy1 = jnp.maximum(y1c, y1r)
    xx2 = jnp.minimum(x2c, x2r)
    yy2 = jnp.minimum(y2c, y2r)
    w = jnp.maximum(jnp.float32(0.0), xx2 - xx1)
    h = jnp.maximum(jnp.float32(0.0), yy2 - yy1)
    inter = w * h
    return inter / (ac + ar - inter + jnp.float32(1e-12))


def _nms_body(x1c, y1c, x2c, y2c, x1r, y1r, x2r, y2r, vr, t_ref,
              kept_ref, sup_ref):
    t = t_ref[...]
    sup_ref[...] = jnp.zeros((NB, 1, B), jnp.float32)
    kept_ref[...] = jnp.zeros((1, NB, 1, B), jnp.float32)
    nvalid = jnp.sum(vr[...]).astype(jnp.int32)
    nb_eff = (nvalid + (B - 1)) // B
    tri = (
        lax.broadcasted_iota(jnp.int32, (B, B), 0)
        < lax.broadcasted_iota(jnp.int32, (B, B), 1)
    ).astype(jnp.float32)

    def blk(k, carry):
        rb = pl.ds(pl.multiple_of(k * B, B), B)
        X1 = x1c[0, rb, :]
        Y1 = y1c[0, rb, :]
        X2 = x2c[0, rb, :]
        Y2 = y2c[0, rb, :]
        AC = (X2 - X1) * (Y2 - Y1)
        x1rb = x1r[0, k]
        y1rb = y1r[0, k]
        x2rb = x2r[0, k]
        y2rb = y2r[0, k]
        AR = (x2rb - x1rb) * (y2rb - y1rb)
        iou = _iou_mat(X1, Y1, X2, Y2, AC, x1rb, y1rb, x2rb, y2rb, AR)
        At = (iou > t).astype(jnp.float32) * tri
        cand = vr[0, k] * (sup_ref[k] < jnp.float32(0.5)).astype(jnp.float32)

        def cond(s):
            return s[1]

        def fbody(s):
            kp, _ = s
            supv = jnp.dot(kp, At, preferred_element_type=jnp.float32)
            mid = cand * (supv < jnp.float32(0.5)).astype(jnp.float32)
            supv2 = jnp.dot(mid, At, preferred_element_type=jnp.float32)
            new = cand * (supv2 < jnp.float32(0.5)).astype(jnp.float32)
            return (new, jnp.any(new != kp))

        keep, _ = lax.while_loop(cond, fbody, (cand, True))
        kept_ref[0, k] = keep

        def cblk(c, carry2):
            x1rc = x1r[0, c]
            y1rc = y1r[0, c]
            x2rc = x2r[0, c]
            y2rc = y2r[0, c]
            ARc = (x2rc - x1rc) * (y2rc - y1rc)
            iou2 = _iou_mat(X1, Y1, X2, Y2, AC, x1rc, y1rc, x2rc, y2rc, ARc)
            supv = jnp.dot(keep, (iou2 > t).astype(jnp.float32),
                           preferred_element_type=jnp.float32)
            sup_ref[c] = jnp.maximum(sup_ref[c], jnp.minimum(supv, 1.0))
            return carry2

        lax.fori_loop(k + 1, nb_eff, cblk, 0)
        return carry

    lax.fori_loop(0, nb_eff, blk, 0)


_SC_CHUNK = 4 * M2 // NW
_SC_KR = _SC_CHUNK // 128
_OUT_ROWS = 20224


def _sc_scatter_body(vals_hbm, idx_hbm, out_hbm, vals_v, idx_v, sem):
    c = lax.axis_index("c")
    s = lax.axis_index("s")
    wid = s * 2 + c
    base = wid * _SC_CHUNK
    pltpu.sync_copy(vals_hbm.at[pl.ds(base, _SC_CHUNK)], vals_v)
    pltpu.sync_copy(idx_hbm.at[wid], idx_v)
    copies = [
        pltpu.async_copy(
            vals_v.at[pl.ds(j * 128, 128)], out_hbm.at[idx_v.at[j]], sem
        )
        for j in range(_SC_KR)
    ]
    for cp in copies:
        cp.wait()


def _sc_scatter(vals, idx):
    mesh = plsc.VectorSubcoreMesh(core_axis_name="c", subcore_axis_name="s")
    return pl.kernel(
        _sc_scatter_body,
        mesh=mesh,
        out_type=jax.ShapeDtypeStruct((_OUT_ROWS, 8), jnp.int32),
        scratch_types=[
            pltpu.VMEM((_SC_CHUNK, 8), jnp.int32),
            pltpu.VMEM((_SC_KR, 128), jnp.int32),
            pltpu.SemaphoreType.DMA,
        ],
        compiler_params=pltpu.CompilerParams(use_tc_tiling_on_sc=False),
    )(vals, idx)


def kernel(b_coords, b_o, b_scores, rpn_nms_thresh=0.7, box_score_thresh=0.05):
    label_dtype = jnp.asarray(np.zeros((), np.int64)).dtype
    N, M, C = b_scores.shape
    pad = M2 - M

    coords_t = jnp.pad(jnp.transpose(b_coords, (0, 2, 1)), ((0, 0), (0, 0), (0, pad)))
    o_p = jnp.pad(b_o, ((0, 0), (0, pad)))[:, None, :]
    scores_t = jnp.pad(jnp.transpose(b_scores, (0, 2, 1)), ((0, 0), (0, 0), (0, pad)))

    coords4, sc, lb, keys = pl.pallas_call(
        _pre_body,
        grid=(N,),
        in_specs=[
            pl.BlockSpec((1, 4, M2), lambda i: (i, 0, 0)),
            pl.BlockSpec((1, 1, M2), lambda i: (i, 0, 0)),
            pl.BlockSpec((1, C, M2), lambda i: (i, 0, 0)),
        ],
        out_specs=[
            pl.BlockSpec((1, 4, M2), lambda i: (i, 0, 0)),
            pl.BlockSpec((1, 1, M2), lambda i: (i, 0, 0)),
            pl.BlockSpec((1, 1, M2), lambda i: (i, 0, 0)),
            pl.BlockSpec((1, 1, M2), lambda i: (i, 0, 0)),
        ],
        out_shape=[
            jax.ShapeDtypeStruct((N, 4, M2), jnp.float32),
            jax.ShapeDtypeStruct((N, 1, M2), jnp.float32),
            jax.ShapeDtypeStruct((N, 1, M2), jnp.int32),
            jax.ShapeDtypeStruct((N, 1, M2), jnp.float32),
        ],
        interpret=_INTERPRET,
    )(coords_t, o_p, scores_t)

    order = jnp.argsort(keys[:, 0, :], axis=-1, stable=True)
    idx3 = order[:, None, :]
    coords_s = jnp.take_along_axis(coords4, idx3, axis=-1)
    sc_s = jnp.take_along_axis(sc, idx3, axis=-1)[:, 0, :]
    lb_s = jnp.take_along_axis(lb, idx3, axis=-1)[:, 0, :]
    keys_s = jnp.take_along_axis(keys, idx3, axis=-1)[:, 0, :]
    valid_s = jnp.isfinite(keys_s).astype(jnp.float32).reshape(N, NB, 1, B)

    x1c = coords_s[:, 0, :, None]
    y1c = coords_s[:, 1, :, None]
    x2c = coords_s[:, 2, :, None]
    y2c = coords_s[:, 3, :, None]
    x1r = coords_s[:, 0, :].reshape(N, NB, 1, B)
    y1r = coords_s[:, 1, :].reshape(N, NB, 1, B)
    x2r = coords_s[:, 2, :].reshape(N, NB, 1, B)
    y2r = coords_s[:, 3, :].reshape(N, NB, 1, B)
    t_arr = jnp.asarray(rpn_nms_thresh, jnp.float32).reshape(1, 1)

    col_spec = pl.BlockSpec((1, M2, 1), lambda i: (i, 0, 0))
    row_spec = pl.BlockSpec((1, NB, 1, B), lambda i: (i, 0, 0, 0))
    kept = pl.pallas_call(
        _nms_body,
        grid=(N,),
        in_specs=[col_spec] * 4
        + [row_spec] * 4
        + [row_spec, pl.BlockSpec((1, 1), lambda i: (0, 0))],
        out_specs=pl.BlockSpec((1, NB, 1, B), lambda i: (i, 0, 0, 0)),
        out_shape=jax.ShapeDtypeStruct((N, NB, 1, B), jnp.float32),
        scratch_shapes=[pltpu.VMEM((NB, 1, B), jnp.float32)],
        interpret=_INTERPRET,
    )(x1c, y1c, x2c, y2c, x1r, y1r, x2r, y2r, valid_s, t_arr)

    keptb = kept.reshape(N, M2) > jnp.float32(0.5)
    seli = (keptb & (sc_s >= box_score_thresh)).astype(jnp.int32)
    pos = jnp.cumsum(seli, axis=-1) - 1
    cnt = jnp.sum(seli, axis=-1, keepdims=True)
    rank_ns = jnp.cumsum(1 - seli, axis=-1) - 1
    tgt_in = jnp.where(seli == 1, pos, cnt + rank_ns)
    base = (jnp.arange(N) * M)[:, None]
    dump = jnp.int32(N * M)
    tgt_g = jnp.where(tgt_in < M, tgt_in + base, dump).astype(jnp.int32)
    idx_sc = tgt_g.reshape(NW, _SC_KR, 128)

    boxes_s = jnp.transpose(coords_s, (0, 2, 1))
    packed = jnp.concatenate(
        [
            lax.bitcast_convert_type(boxes_s, jnp.int32),
            lax.bitcast_convert_type(sc_s, jnp.int32)[..., None],
            lb_s[..., None],
            jnp.zeros((N, M2, 2), jnp.int32),
        ],
        axis=-1,
    )
    packed = jnp.where((seli == 1)[..., None], packed, 0).reshape(N * M2, 8)

    out_flat = _sc_scatter(packed, idx_sc)
    body = out_flat[: N * M]
    out_boxes = lax.bitcast_convert_type(body[:, :4], jnp.float32).reshape(N, M, 4)
    out_scores = lax.bitcast_convert_type(body[:, 4], jnp.float32).reshape(N, M)
    out_labels = body[:, 5].astype(label_dtype).reshape(N, M)
    counts = cnt[:, 0].astype(jnp.int32)
    return (out_boxes, out_scores, out_labels, counts)


# speedup vs baseline: 1.0091x; 1.0091x over previous
import functools
import numpy as np
import jax
import jax.numpy as jnp
from jax import lax
from jax.experimental import pallas as pl
from jax.experimental.pallas import tpu as pltpu
from jax.experimental.pallas import tpu_sc as plsc

_INTERPRET = False

M2 = 5120
B = 512
NB = M2 // B
NW = 32


def _pre_body(ct_ref, o_ref, st_ref, co_ref, sc_ref, lb_ref, ky_ref):
    x = ct_ref[0, 0:1, :]
    y = ct_ref[0, 1:2, :]
    w = ct_ref[0, 2:3, :]
    h = ct_ref[0, 3:4, :]
    one = jnp.float32(1.0)
    zero = jnp.float32(0.0)
    x1 = jnp.clip(x, zero, one)
    y1 = jnp.clip(y, zero, one)
    x2 = jnp.clip(x + w, zero, one)
    y2 = jnp.clip(y + h, zero, one)
    co_ref[0, 0:1, :] = x1
    co_ref[0, 1:2, :] = y1
    co_ref[0, 2:3, :] = x2
    co_ref[0, 3:4, :] = y2
    C = st_ref.shape[1]
    best = st_ref[0, 0:1, :]
    bi = jnp.zeros(best.shape, jnp.int32)
    for c in range(1, C):
        s = st_ref[0, c : c + 1, :]
        gt = s > best
        best = jnp.where(gt, s, best)
        bi = jnp.where(gt, jnp.int32(c), bi)
    sc_ref[0] = best
    lb_ref[0] = bi
    valid = o_ref[0] >= jnp.float32(0.5)
    ky_ref[0] = jnp.where(valid, -best, jnp.float32(jnp.inf))


def _iou_mat(x1c, y1c, x2c, y2c, ac, x1r, y1r, x2r, y2r, ar):
    xx1 = jnp.maximum(x1c, x1r)
    yy1 = jnp.maximum(y1c, y1r)
    xx2 = jnp.minimum(x2c, x2r)
    yy2 = jnp.minimum(y2c, y2r)
    w = jnp.maximum(jnp.float32(0.0), xx2 - xx1)
    h = jnp.maximum(jnp.float32(0.0), yy2 - yy1)
    inter = w * h
    return inter / (ac + ar - inter + jnp.float32(1e-12))


def _nms_body(x1c, y1c, x2c, y2c, x1r, y1r, x2r, y2r, vr, t_ref,
              kept_ref, sup_ref):
    t = t_ref[...]
    sup_ref[...] = jnp.zeros((NB, 1, B), jnp.float32)
    kept_ref[...] = jnp.zeros((1, NB, 1, B), jnp.float32)
    nvalid = jnp.sum(vr[...]).astype(jnp.int32)
    nb_eff = (nvalid + (B - 1)) // B
    tri = (
        lax.broadcasted_iota(jnp.int32, (B, B), 0)
        < lax.broadcasted_iota(jnp.int32, (B, B), 1)
    ).astype(jnp.float32)

    def blk(k, carry):
        rb = pl.ds(pl.multiple_of(k * B, B), B)
        X1 = x1c[0, rb, :]
        Y1 = y1c[0, rb, :]
        X2 = x2c[0, rb, :]
        Y2 = y2c[0, rb, :]
        AC = (X2 - X1) * (Y2 - Y1)
        x1rb = x1r[0, k]
        y1rb = y1r[0, k]
        x2rb = x2r[0, k]
        y2rb = y2r[0, k]
        AR = (x2rb - x1rb) * (y2rb - y1rb)
        iou = _iou_mat(X1, Y1, X2, Y2, AC, x1rb, y1rb, x2rb, y2rb, AR)
        At = (iou > t).astype(jnp.float32) * tri
        cand = vr[0, k] * (sup_ref[k] < jnp.float32(0.5)).astype(jnp.float32)

        def cond(s):
            return s[1]

        def fbody(s):
            kp, _ = s
            supv = jnp.dot(kp, At, preferred_element_type=jnp.float32)
            new = cand * (supv < jnp.float32(0.5)).astype(jnp.float32)
            return (new, jnp.any(new != kp))

        keep, _ = lax.while_loop(cond, fbody, (cand, True))
        kept_ref[0, k] = keep

        def cblk(c, carry2):
            x1rc = x1r[0, c]
            y1rc = y1r[0, c]
            x2rc = x2r[0, c]
            y2rc = y2r[0, c]
            ARc = (x2rc - x1rc) * (y2rc - y1rc)
            iou2 = _iou_mat(X1, Y1, X2, Y2, AC, x1rc, y1rc, x2rc, y2rc, ARc)
            supv = jnp.dot(keep, (iou2 > t).astype(jnp.float32),
                           preferred_element_type=jnp.float32)
            sup_ref[c] = jnp.maximum(sup_ref[c], jnp.minimum(supv, 1.0))
            return carry2

        lax.fori_loop(k + 1, nb_eff, cblk, 0)
        return carry

    lax.fori_loop(0, nb_eff, blk, 0)


_SC_CHUNK = 4 * M2 // NW
_SC_KR = _SC_CHUNK // 128
_OUT_ROWS = 20224


def _sc_scatter_body(vals_hbm, idx_hbm, out_hbm, vals_v, idx_v, sem):
    c = lax.axis_index("c")
    s = lax.axis_index("s")
    wid = s * 2 + c
    base = wid * _SC_CHUNK
    pltpu.sync_copy(vals_hbm.at[pl.ds(base, _SC_CHUNK)], vals_v)
    pltpu.sync_copy(idx_hbm.at[wid], idx_v)
    copies = [
        pltpu.async_copy(
            vals_v.at[pl.ds(j * 128, 128)], out_hbm.at[idx_v.at[j]], sem
        )
        for j in range(_SC_KR)
    ]
    for cp in copies:
        cp.wait()


def _sc_scatter(vals, idx):
    mesh = plsc.VectorSubcoreMesh(core_axis_name="c", subcore_axis_name="s")
    return pl.kernel(
        _sc_scatter_body,
        mesh=mesh,
        out_type=jax.ShapeDtypeStruct((_OUT_ROWS, 8), jnp.int32),
        scratch_types=[
            pltpu.VMEM((_SC_CHUNK, 8), jnp.int32),
            pltpu.VMEM((_SC_KR, 128), jnp.int32),
            pltpu.SemaphoreType.DMA,
        ],
        compiler_params=pltpu.CompilerParams(use_tc_tiling_on_sc=False),
    )(vals, idx)


def kernel(b_coords, b_o, b_scores, rpn_nms_thresh=0.7, box_score_thresh=0.05):
    label_dtype = jnp.asarray(np.zeros((), np.int64)).dtype
    N, M, C = b_scores.shape
    pad = M2 - M

    coords_t = jnp.pad(jnp.transpose(b_coords, (0, 2, 1)), ((0, 0), (0, 0), (0, pad)))
    o_p = jnp.pad(b_o, ((0, 0), (0, pad)))[:, None, :]
    scores_t = jnp.pad(jnp.transpose(b_scores, (0, 2, 1)), ((0, 0), (0, 0), (0, pad)))

    coords4, sc, lb, keys = pl.pallas_call(
        _pre_body,
        grid=(N,),
        in_specs=[
            pl.BlockSpec((1, 4, M2), lambda i: (i, 0, 0)),
            pl.BlockSpec((1, 1, M2), lambda i: (i, 0, 0)),
            pl.BlockSpec((1, C, M2), lambda i: (i, 0, 0)),
        ],
        out_specs=[
            pl.BlockSpec((1, 4, M2), lambda i: (i, 0, 0)),
            pl.BlockSpec((1, 1, M2), lambda i: (i, 0, 0)),
            pl.BlockSpec((1, 1, M2), lambda i: (i, 0, 0)),
            pl.BlockSpec((1, 1, M2), lambda i: (i, 0, 0)),
        ],
        out_shape=[
            jax.ShapeDtypeStruct((N, 4, M2), jnp.float32),
            jax.ShapeDtypeStruct((N, 1, M2), jnp.float32),
            jax.ShapeDtypeStruct((N, 1, M2), jnp.int32),
            jax.ShapeDtypeStruct((N, 1, M2), jnp.float32),
        ],
        interpret=_INTERPRET,
    )(coords_t, o_p, scores_t)

    order = jnp.argsort(keys[:, 0, :], axis=-1, stable=True)
    idx3 = order[:, None, :]
    coords_s = jnp.take_along_axis(coords4, idx3, axis=-1)
    sc_s = jnp.take_along_axis(sc, idx3, axis=-1)[:, 0, :]
    lb_s = jnp.take_along_axis(lb, idx3, axis=-1)[:, 0, :]
    keys_s = jnp.take_along_axis(keys, idx3, axis=-1)[:, 0, :]
    valid_s = jnp.isfinite(keys_s).astype(jnp.float32).reshape(N, NB, 1, B)

    x1c = coords_s[:, 0, :, None]
    y1c = coords_s[:, 1, :, None]
    x2c = coords_s[:, 2, :, None]
    y2c = coords_s[:, 3, :, None]
    x1r = coords_s[:, 0, :].reshape(N, NB, 1, B)
    y1r = coords_s[:, 1, :].reshape(N, NB, 1, B)
    x2r = coords_s[:, 2, :].reshape(N, NB, 1, B)
    y2r = coords_s[:, 3, :].reshape(N, NB, 1, B)
    t_arr = jnp.asarray(rpn_nms_thresh, jnp.float32).reshape(1, 1)

    col_spec = pl.BlockSpec((1, M2, 1), lambda i: (i, 0, 0))
    row_spec = pl.BlockSpec((1, NB, 1, B), lambda i: (i, 0, 0, 0))
    kept = pl.pallas_call(
        _nms_body,
        grid=(N,),
        in_specs=[col_spec] * 4
        + [row_spec] * 4
        + [row_spec, pl.BlockSpec((1, 1), lambda i: (0, 0))],
        out_specs=pl.BlockSpec((1, NB, 1, B), lambda i: (i, 0, 0, 0)),
        out_shape=jax.ShapeDtypeStruct((N, NB, 1, B), jnp.float32),
        scratch_shapes=[pltpu.VMEM((NB, 1, B), jnp.float32)],
        interpret=_INTERPRET,
    )(x1c, y1c, x2c, y2c, x1r, y1r, x2r, y2r, valid_s, t_arr)

    keptb = kept.reshape(N, M2) > jnp.float32(0.5)
    seli = (keptb & (sc_s >= box_score_thresh)).astype(jnp.int32)
    pos = jnp.cumsum(seli, axis=-1) - 1
    cnt = jnp.sum(seli, axis=-1, keepdims=True)
    rank_ns = jnp.cumsum(1 - seli, axis=-1) - 1
    tgt_in = jnp.where(seli == 1, pos, cnt + rank_ns)
    base = (jnp.arange(N) * M)[:, None]
    dump = jnp.int32(N * M)
    tgt_g = jnp.where(tgt_in < M, tgt_in + base, dump).astype(jnp.int32)
    idx_sc = tgt_g.reshape(NW, _SC_KR, 128)

    boxes_s = jnp.transpose(coords_s, (0, 2, 1))
    packed = jnp.concatenate(
        [
            lax.bitcast_convert_type(boxes_s, jnp.int32),
            lax.bitcast_convert_type(sc_s, jnp.int32)[..., None],
            lb_s[..., None],
            jnp.zeros((N, M2, 2), jnp.int32),
        ],
        axis=-1,
    )
    packed = jnp.where((seli == 1)[..., None], packed, 0).reshape(N * M2, 8)

    out_flat = _sc_scatter(packed, idx_sc)
    body = out_flat[: N * M]
    out_boxes = lax.bitcast_convert_type(body[:, :4], jnp.float32).reshape(N, M, 4)
    out_scores = lax.bitcast_convert_type(body[:, 4], jnp.float32).reshape(N, M)
    out_labels = body[:, 5].astype(label_dtype).reshape(N, M)
    counts = cnt[:, 0].astype(jnp.int32)
    return (out_boxes, out_scores, out_labels, counts)


# speedup vs baseline: 1.1029x; 1.0929x over previous
import functools
import numpy as np
import jax
import jax.numpy as jnp
from jax import lax
from jax.experimental import pallas as pl
from jax.experimental.pallas import tpu as pltpu
from jax.experimental.pallas import tpu_sc as plsc

_INTERPRET = False

M2 = 5120
B = 512
NB = M2 // B
NW = 32


def _pre_body(ct_ref, o_ref, st_ref, co_ref, sc_ref, lb_ref, ky_ref):
    x = ct_ref[0, 0:1, :]
    y = ct_ref[0, 1:2, :]
    w = ct_ref[0, 2:3, :]
    h = ct_ref[0, 3:4, :]
    one = jnp.float32(1.0)
    zero = jnp.float32(0.0)
    x1 = jnp.clip(x, zero, one)
    y1 = jnp.clip(y, zero, one)
    x2 = jnp.clip(x + w, zero, one)
    y2 = jnp.clip(y + h, zero, one)
    co_ref[0, 0:1, :] = x1
    co_ref[0, 1:2, :] = y1
    co_ref[0, 2:3, :] = x2
    co_ref[0, 3:4, :] = y2
    C = st_ref.shape[1]
    best = st_ref[0, 0:1, :]
    bi = jnp.zeros(best.shape, jnp.int32)
    for c in range(1, C):
        s = st_ref[0, c : c + 1, :]
        gt = s > best
        best = jnp.where(gt, s, best)
        bi = jnp.where(gt, jnp.int32(c), bi)
    sc_ref[0] = best
    lb_ref[0] = bi
    valid = o_ref[0] >= jnp.float32(0.5)
    ky_ref[0] = jnp.where(valid, -best, jnp.float32(jnp.inf))


def _iou_mat(x1c, y1c, x2c, y2c, ac, x1r, y1r, x2r, y2r, ar):
    xx1 = jnp.maximum(x1c, x1r)
    yy1 = jnp.maximum(y1c, y1r)
    xx2 = jnp.minimum(x2c, x2r)
    yy2 = jnp.minimum(y2c, y2r)
    w = jnp.maximum(jnp.float32(0.0), xx2 - xx1)
    h = jnp.maximum(jnp.float32(0.0), yy2 - yy1)
    inter = w * h
    return inter / (ac + ar - inter + jnp.float32(1e-12))


def _nms_body(x1c, y1c, x2c, y2c, x1r, y1r, x2r, y2r, vr, t_ref,
              kept_ref, sup_ref):
    t = t_ref[...]
    sup_ref[...] = jnp.zeros((1, M2), jnp.float32)
    kept_ref[...] = jnp.zeros((1, 1, M2), jnp.float32)
    nvalid = jnp.sum(vr[...]).astype(jnp.int32)
    nb_eff = (nvalid + (B - 1)) // B
    tri = (
        lax.broadcasted_iota(jnp.int32, (B, B), 0)
        < lax.broadcasted_iota(jnp.int32, (B, B), 1)
    ).astype(jnp.float32)

    def blk(k, carry):
        rb = pl.ds(pl.multiple_of(k * B, B), B)
        X1 = x1c[0, rb, :]
        Y1 = y1c[0, rb, :]
        X2 = x2c[0, rb, :]
        Y2 = y2c[0, rb, :]
        AC = (X2 - X1) * (Y2 - Y1)
        x1rb = x1r[0, :, rb]
        y1rb = y1r[0, :, rb]
        x2rb = x2r[0, :, rb]
        y2rb = y2r[0, :, rb]
        AR = (x2rb - x1rb) * (y2rb - y1rb)
        iou = _iou_mat(X1, Y1, X2, Y2, AC, x1rb, y1rb, x2rb, y2rb, AR)
        At = (iou > t).astype(jnp.float32) * tri
        cand = vr[0, :, rb] * (sup_ref[:, rb] < jnp.float32(0.5)).astype(jnp.float32)

        def cond(s):
            return s[1]

        def fbody(s):
            kp, _ = s
            supv = jnp.dot(kp, At, preferred_element_type=jnp.float32)
            new = cand * (supv < jnp.float32(0.5)).astype(jnp.float32)
            return (new, jnp.any(new != kp))

        keep, _ = lax.while_loop(cond, fbody, (cand, True))
        kept_ref[0, :, rb] = keep

        def cblk(c, carry2):
            rc = pl.ds(pl.multiple_of(c * B, B), B)
            x1rc = x1r[0, :, rc]
            y1rc = y1r[0, :, rc]
            x2rc = x2r[0, :, rc]
            y2rc = y2r[0, :, rc]
            ARc = (x2rc - x1rc) * (y2rc - y1rc)
            iou2 = _iou_mat(X1, Y1, X2, Y2, AC, x1rc, y1rc, x2rc, y2rc, ARc)
            supv = jnp.dot(keep, (iou2 > t).astype(jnp.float32),
                           preferred_element_type=jnp.float32)
            sup_ref[:, rc] = jnp.maximum(sup_ref[:, rc], jnp.minimum(supv, 1.0))
            return carry2

        lax.fori_loop(k + 1, nb_eff, cblk, 0)
        return carry

    lax.fori_loop(0, nb_eff, blk, 0)


_SC_CHUNK = 4 * M2 // NW
_SC_KR = _SC_CHUNK // 128
_OUT_ROWS = 20224


def _sc_gather_body(table_hbm, idx_hbm, out_hbm, vals_v, idx_v, sem):
    c = lax.axis_index("c")
    s = lax.axis_index("s")
    wid = s * 2 + c
    base = wid * _SC_CHUNK
    pltpu.sync_copy(idx_hbm.at[wid], idx_v)
    copies = [
        pltpu.async_copy(
            table_hbm.at[idx_v.at[j]], vals_v.at[pl.ds(j * 128, 128)], sem
        )
        for j in range(_SC_KR)
    ]
    for cp in copies:
        cp.wait()
    pltpu.sync_copy(vals_v, out_hbm.at[pl.ds(base, _SC_CHUNK)])


def _sc_gather(table, idx):
    mesh = plsc.VectorSubcoreMesh(core_axis_name="c", subcore_axis_name="s")
    return pl.kernel(
        _sc_gather_body,
        mesh=mesh,
        out_type=jax.ShapeDtypeStruct((4 * M2, 8), jnp.int32),
        scratch_types=[
            pltpu.VMEM((_SC_CHUNK, 8), jnp.int32),
            pltpu.VMEM((_SC_KR, 128), jnp.int32),
            pltpu.SemaphoreType.DMA,
        ],
        compiler_params=pltpu.CompilerParams(use_tc_tiling_on_sc=False),
    )(table, idx)


def _sc_scatter_body(vals_hbm, idx_hbm, out_hbm, vals_v, idx_v, sem):
    c = lax.axis_index("c")
    s = lax.axis_index("s")
    wid = s * 2 + c
    base = wid * _SC_CHUNK
    pltpu.sync_copy(vals_hbm.at[pl.ds(base, _SC_CHUNK)], vals_v)
    pltpu.sync_copy(idx_hbm.at[wid], idx_v)
    copies = [
        pltpu.async_copy(
            vals_v.at[pl.ds(j * 128, 128)], out_hbm.at[idx_v.at[j]], sem
        )
        for j in range(_SC_KR)
    ]
    for cp in copies:
        cp.wait()


def _sc_scatter(vals, idx):
    mesh = plsc.VectorSubcoreMesh(core_axis_name="c", subcore_axis_name="s")
    return pl.kernel(
        _sc_scatter_body,
        mesh=mesh,
        out_type=jax.ShapeDtypeStruct((_OUT_ROWS, 8), jnp.int32),
        scratch_types=[
            pltpu.VMEM((_SC_CHUNK, 8), jnp.int32),
            pltpu.VMEM((_SC_KR, 128), jnp.int32),
            pltpu.SemaphoreType.DMA,
        ],
        compiler_params=pltpu.CompilerParams(use_tc_tiling_on_sc=False),
    )(vals, idx)


def kernel(b_coords, b_o, b_scores, rpn_nms_thresh=0.7, box_score_thresh=0.05):
    label_dtype = jnp.asarray(np.zeros((), np.int64)).dtype
    N, M, C = b_scores.shape
    pad = M2 - M

    coords_t = jnp.pad(jnp.transpose(b_coords, (0, 2, 1)), ((0, 0), (0, 0), (0, pad)))
    o_p = jnp.pad(b_o, ((0, 0), (0, pad)))[:, None, :]
    scores_t = jnp.pad(jnp.transpose(b_scores, (0, 2, 1)), ((0, 0), (0, 0), (0, pad)))

    coords4, sc, lb, keys = pl.pallas_call(
        _pre_body,
        grid=(N,),
        in_specs=[
            pl.BlockSpec((1, 4, M2), lambda i: (i, 0, 0)),
            pl.BlockSpec((1, 1, M2), lambda i: (i, 0, 0)),
            pl.BlockSpec((1, C, M2), lambda i: (i, 0, 0)),
        ],
        out_specs=[
            pl.BlockSpec((1, 4, M2), lambda i: (i, 0, 0)),
            pl.BlockSpec((1, 1, M2), lambda i: (i, 0, 0)),
            pl.BlockSpec((1, 1, M2), lambda i: (i, 0, 0)),
            pl.BlockSpec((1, 1, M2), lambda i: (i, 0, 0)),
        ],
        out_shape=[
            jax.ShapeDtypeStruct((N, 4, M2), jnp.float32),
            jax.ShapeDtypeStruct((N, 1, M2), jnp.float32),
            jax.ShapeDtypeStruct((N, 1, M2), jnp.int32),
            jax.ShapeDtypeStruct((N, 1, M2), jnp.float32),
        ],
        interpret=_INTERPRET,
    )(coords_t, o_p, scores_t)

    order = jnp.argsort(keys[:, 0, :], axis=-1, stable=True)

    packed_u = jnp.concatenate(
        [
            lax.bitcast_convert_type(
                jnp.transpose(coords4, (0, 2, 1)), jnp.int32
            ),
            lax.bitcast_convert_type(sc[:, 0, :], jnp.int32)[..., None],
            lb[:, 0, :, None],
            jnp.zeros((N, M2, 2), jnp.int32),
        ],
        axis=-1,
    ).reshape(N * M2, 8)
    gidx = (order + (jnp.arange(N) * M2)[:, None]).astype(jnp.int32)
    packed_s = _sc_gather(packed_u, gidx.reshape(NW, _SC_KR, 128))
    packed_s = packed_s.reshape(N, M2, 8)

    coords_s = jnp.transpose(
        lax.bitcast_convert_type(packed_s[:, :, :4], jnp.float32), (0, 2, 1)
    )
    sc_s = lax.bitcast_convert_type(packed_s[:, :, 4], jnp.float32)
    lb_s = packed_s[:, :, 5]
    nval = jnp.sum(jnp.isfinite(keys[:, 0, :]), axis=-1, dtype=jnp.int32)
    valid_s = (
        jnp.arange(M2, dtype=jnp.int32)[None, :] < nval[:, None]
    ).astype(jnp.float32)[:, None, :]

    x1c = coords_s[:, 0, :, None]
    y1c = coords_s[:, 1, :, None]
    x2c = coords_s[:, 2, :, None]
    y2c = coords_s[:, 3, :, None]
    x1r = coords_s[:, 0:1, :]
    y1r = coords_s[:, 1:2, :]
    x2r = coords_s[:, 2:3, :]
    y2r = coords_s[:, 3:4, :]
    t_arr = jnp.asarray(rpn_nms_thresh, jnp.float32).reshape(1, 1)

    col_spec = pl.BlockSpec((1, M2, 1), lambda i: (i, 0, 0))
    row_spec = pl.BlockSpec((1, 1, M2), lambda i: (i, 0, 0))
    kept = pl.pallas_call(
        _nms_body,
        grid=(N,),
        in_specs=[col_spec] * 4
        + [row_spec] * 4
        + [row_spec, pl.BlockSpec((1, 1), lambda i: (0, 0))],
        out_specs=pl.BlockSpec((1, 1, M2), lambda i: (i, 0, 0)),
        out_shape=jax.ShapeDtypeStruct((N, 1, M2), jnp.float32),
        scratch_shapes=[pltpu.VMEM((1, M2), jnp.float32)],
        interpret=_INTERPRET,
    )(x1c, y1c, x2c, y2c, x1r, y1r, x2r, y2r, valid_s, t_arr)

    keptb = kept[:, 0, :] > jnp.float32(0.5)
    seli = (keptb & (sc_s >= box_score_thresh)).astype(jnp.int32)
    pos = jnp.cumsum(seli, axis=-1) - 1
    cnt = jnp.sum(seli, axis=-1, keepdims=True)
    rank_ns = jnp.cumsum(1 - seli, axis=-1) - 1
    tgt_in = jnp.where(seli == 1, pos, cnt + rank_ns)
    base = (jnp.arange(N) * M)[:, None]
    dump = jnp.int32(N * M)
    tgt_g = jnp.where(tgt_in < M, tgt_in + base, dump).astype(jnp.int32)
    idx_sc = tgt_g.reshape(NW, _SC_KR, 128)

    packed = jnp.where((seli == 1)[..., None], packed_s, 0).reshape(N * M2, 8)

    out_flat = _sc_scatter(packed, idx_sc)
    body = out_flat[: N * M]
    out_boxes = lax.bitcast_convert_type(body[:, :4], jnp.float32).reshape(N, M, 4)
    out_scores = lax.bitcast_convert_type(body[:, 4], jnp.float32).reshape(N, M)
    out_labels = body[:, 5].astype(label_dtype).reshape(N, M)
    counts = cnt[:, 0].astype(jnp.int32)
    return (out_boxes, out_scores, out_labels, counts)


# speedup vs baseline: 1.1045x; 1.0015x over previous
import functools
import numpy as np
import jax
import jax.numpy as jnp
from jax import lax
from jax.experimental import pallas as pl
from jax.experimental.pallas import tpu as pltpu
from jax.experimental.pallas import tpu_sc as plsc

_INTERPRET = False

M2 = 5120
B = 512
NB = M2 // B
NW = 32


def _pre_body(ct_ref, o_ref, st_ref, co_ref, sc_ref, lb_ref, ky_ref):
    x = ct_ref[0, 0:1, :]
    y = ct_ref[0, 1:2, :]
    w = ct_ref[0, 2:3, :]
    h = ct_ref[0, 3:4, :]
    one = jnp.float32(1.0)
    zero = jnp.float32(0.0)
    x1 = jnp.clip(x, zero, one)
    y1 = jnp.clip(y, zero, one)
    x2 = jnp.clip(x + w, zero, one)
    y2 = jnp.clip(y + h, zero, one)
    co_ref[0, 0:1, :] = x1
    co_ref[0, 1:2, :] = y1
    co_ref[0, 2:3, :] = x2
    co_ref[0, 3:4, :] = y2
    C = st_ref.shape[1]
    best = st_ref[0, 0:1, :]
    bi = jnp.zeros(best.shape, jnp.int32)
    for c in range(1, C):
        s = st_ref[0, c : c + 1, :]
        gt = s > best
        best = jnp.where(gt, s, best)
        bi = jnp.where(gt, jnp.int32(c), bi)
    sc_ref[0] = best
    lb_ref[0] = bi
    valid = o_ref[0] >= jnp.float32(0.5)
    ky_ref[0] = jnp.where(valid, -best, jnp.float32(jnp.inf))


def _iou_mat(x1c, y1c, x2c, y2c, ac, x1r, y1r, x2r, y2r, ar):
    xx1 = jnp.maximum(x1c, x1r)
    yy1 = jnp.maximum(y1c, y1r)
    xx2 = jnp.minimum(x2c, x2r)
    yy2 = jnp.minimum(y2c, y2r)
    w = jnp.maximum(jnp.float32(0.0), xx2 - xx1)
    h = jnp.maximum(jnp.float32(0.0), yy2 - yy1)
    inter = w * h
    return inter / (ac + ar - inter + jnp.float32(1e-12))


def _nms_body(x1c, y1c, x2c, y2c, x1r, y1r, x2r, y2r, vr, t_ref,
              kept_ref, sup_ref):
    t = t_ref[...]
    sup_ref[...] = jnp.zeros((1, M2), jnp.float32)
    kept_ref[...] = jnp.zeros((1, 1, M2), jnp.float32)
    nvalid = jnp.sum(vr[...]).astype(jnp.int32)
    nb_eff = (nvalid + (B - 1)) // B
    tri = (
        lax.broadcasted_iota(jnp.int32, (B, B), 0)
        < lax.broadcasted_iota(jnp.int32, (B, B), 1)
    ).astype(jnp.float32)

    def blk(k, carry):
        rb = pl.ds(pl.multiple_of(k * B, B), B)
        X1 = x1c[0, rb, :]
        Y1 = y1c[0, rb, :]
        X2 = x2c[0, rb, :]
        Y2 = y2c[0, rb, :]
        AC = (X2 - X1) * (Y2 - Y1)
        x1rb = x1r[0, :, rb]
        y1rb = y1r[0, :, rb]
        x2rb = x2r[0, :, rb]
        y2rb = y2r[0, :, rb]
        AR = (x2rb - x1rb) * (y2rb - y1rb)
        iou = _iou_mat(X1, Y1, X2, Y2, AC, x1rb, y1rb, x2rb, y2rb, AR)
        At = ((iou > t).astype(jnp.float32) * tri).astype(jnp.bfloat16)
        cand = vr[0, :, rb] * (sup_ref[:, rb] < jnp.float32(0.5)).astype(jnp.float32)

        def cond(s):
            return s[1]

        def fbody(s):
            kp, _ = s
            supv = jnp.dot(kp.astype(jnp.bfloat16), At,
                           preferred_element_type=jnp.float32)
            new = cand * (supv < jnp.float32(0.5)).astype(jnp.float32)
            return (new, jnp.any(new != kp))

        keep, _ = lax.while_loop(cond, fbody, (cand, True))
        kept_ref[0, :, rb] = keep
        keep16 = keep.astype(jnp.bfloat16)

        def cblk(c, carry2):
            rc = pl.ds(pl.multiple_of(c * B, B), B)
            x1rc = x1r[0, :, rc]
            y1rc = y1r[0, :, rc]
            x2rc = x2r[0, :, rc]
            y2rc = y2r[0, :, rc]
            ARc = (x2rc - x1rc) * (y2rc - y1rc)
            iou2 = _iou_mat(X1, Y1, X2, Y2, AC, x1rc, y1rc, x2rc, y2rc, ARc)
            supv = jnp.dot(keep16, (iou2 > t).astype(jnp.bfloat16),
                           preferred_element_type=jnp.float32)
            sup_ref[:, rc] = jnp.maximum(sup_ref[:, rc], jnp.minimum(supv, 1.0))
            return carry2

        lax.fori_loop(k + 1, nb_eff, cblk, 0)
        return carry

    lax.fori_loop(0, nb_eff, blk, 0)


_SC_CHUNK = 4 * M2 // NW
_SC_KR = _SC_CHUNK // 128
_OUT_ROWS = 20224


def _sc_gather_body(table_hbm, idx_hbm, out_hbm, vals_v, idx_v, sem):
    c = lax.axis_index("c")
    s = lax.axis_index("s")
    wid = s * 2 + c
    base = wid * _SC_CHUNK
    pltpu.sync_copy(idx_hbm.at[wid], idx_v)
    copies = [
        pltpu.async_copy(
            table_hbm.at[idx_v.at[j]], vals_v.at[pl.ds(j * 128, 128)], sem
        )
        for j in range(_SC_KR)
    ]
    for cp in copies:
        cp.wait()
    pltpu.sync_copy(vals_v, out_hbm.at[pl.ds(base, _SC_CHUNK)])


def _sc_gather(table, idx):
    mesh = plsc.VectorSubcoreMesh(core_axis_name="c", subcore_axis_name="s")
    return pl.kernel(
        _sc_gather_body,
        mesh=mesh,
        out_type=jax.ShapeDtypeStruct((4 * M2, 8), jnp.int32),
        scratch_types=[
            pltpu.VMEM((_SC_CHUNK, 8), jnp.int32),
            pltpu.VMEM((_SC_KR, 128), jnp.int32),
            pltpu.SemaphoreType.DMA,
        ],
        compiler_params=pltpu.CompilerParams(use_tc_tiling_on_sc=False),
    )(table, idx)


def _sc_scatter_body(vals_hbm, idx_hbm, out_hbm, vals_v, idx_v, sem):
    c = lax.axis_index("c")
    s = lax.axis_index("s")
    wid = s * 2 + c
    base = wid * _SC_CHUNK
    pltpu.sync_copy(vals_hbm.at[pl.ds(base, _SC_CHUNK)], vals_v)
    pltpu.sync_copy(idx_hbm.at[wid], idx_v)
    copies = [
        pltpu.async_copy(
            vals_v.at[pl.ds(j * 128, 128)], out_hbm.at[idx_v.at[j]], sem
        )
        for j in range(_SC_KR)
    ]
    for cp in copies:
        cp.wait()


def _sc_scatter(vals, idx):
    mesh = plsc.VectorSubcoreMesh(core_axis_name="c", subcore_axis_name="s")
    return pl.kernel(
        _sc_scatter_body,
        mesh=mesh,
        out_type=jax.ShapeDtypeStruct((_OUT_ROWS, 8), jnp.int32),
        scratch_types=[
            pltpu.VMEM((_SC_CHUNK, 8), jnp.int32),
            pltpu.VMEM((_SC_KR, 128), jnp.int32),
            pltpu.SemaphoreType.DMA,
        ],
        compiler_params=pltpu.CompilerParams(use_tc_tiling_on_sc=False),
    )(vals, idx)


def kernel(b_coords, b_o, b_scores, rpn_nms_thresh=0.7, box_score_thresh=0.05):
    label_dtype = jnp.asarray(np.zeros((), np.int64)).dtype
    N, M, C = b_scores.shape
    pad = M2 - M

    coords_t = jnp.pad(jnp.transpose(b_coords, (0, 2, 1)), ((0, 0), (0, 0), (0, pad)))
    o_p = jnp.pad(b_o, ((0, 0), (0, pad)))[:, None, :]
    scores_t = jnp.pad(jnp.transpose(b_scores, (0, 2, 1)), ((0, 0), (0, 0), (0, pad)))

    coords4, sc, lb, keys = pl.pallas_call(
        _pre_body,
        grid=(N,),
        in_specs=[
            pl.BlockSpec((1, 4, M2), lambda i: (i, 0, 0)),
            pl.BlockSpec((1, 1, M2), lambda i: (i, 0, 0)),
            pl.BlockSpec((1, C, M2), lambda i: (i, 0, 0)),
        ],
        out_specs=[
            pl.BlockSpec((1, 4, M2), lambda i: (i, 0, 0)),
            pl.BlockSpec((1, 1, M2), lambda i: (i, 0, 0)),
            pl.BlockSpec((1, 1, M2), lambda i: (i, 0, 0)),
            pl.BlockSpec((1, 1, M2), lambda i: (i, 0, 0)),
        ],
        out_shape=[
            jax.ShapeDtypeStruct((N, 4, M2), jnp.float32),
            jax.ShapeDtypeStruct((N, 1, M2), jnp.float32),
            jax.ShapeDtypeStruct((N, 1, M2), jnp.int32),
            jax.ShapeDtypeStruct((N, 1, M2), jnp.float32),
        ],
        interpret=_INTERPRET,
    )(coords_t, o_p, scores_t)

    order = jnp.argsort(keys[:, 0, :], axis=-1, stable=True)

    packed_u = jnp.concatenate(
        [
            lax.bitcast_convert_type(
                jnp.transpose(coords4, (0, 2, 1)), jnp.int32
            ),
            lax.bitcast_convert_type(sc[:, 0, :], jnp.int32)[..., None],
            lb[:, 0, :, None],
            jnp.zeros((N, M2, 2), jnp.int32),
        ],
        axis=-1,
    ).reshape(N * M2, 8)
    gidx = (order + (jnp.arange(N) * M2)[:, None]).astype(jnp.int32)
    packed_s = _sc_gather(packed_u, gidx.reshape(NW, _SC_KR, 128))
    packed_s = packed_s.reshape(N, M2, 8)

    coords_s = jnp.transpose(
        lax.bitcast_convert_type(packed_s[:, :, :4], jnp.float32), (0, 2, 1)
    )
    sc_s = lax.bitcast_convert_type(packed_s[:, :, 4], jnp.float32)
    lb_s = packed_s[:, :, 5]
    nval = jnp.sum(jnp.isfinite(keys[:, 0, :]), axis=-1, dtype=jnp.int32)
    valid_s = (
        jnp.arange(M2, dtype=jnp.int32)[None, :] < nval[:, None]
    ).astype(jnp.float32)[:, None, :]

    x1c = coords_s[:, 0, :, None]
    y1c = coords_s[:, 1, :, None]
    x2c = coords_s[:, 2, :, None]
    y2c = coords_s[:, 3, :, None]
    x1r = coords_s[:, 0:1, :]
    y1r = coords_s[:, 1:2, :]
    x2r = coords_s[:, 2:3, :]
    y2r = coords_s[:, 3:4, :]
    t_arr = jnp.asarray(rpn_nms_thresh, jnp.float32).reshape(1, 1)

    col_spec = pl.BlockSpec((1, M2, 1), lambda i: (i, 0, 0))
    row_spec = pl.BlockSpec((1, 1, M2), lambda i: (i, 0, 0))
    kept = pl.pallas_call(
        _nms_body,
        grid=(N,),
        in_specs=[col_spec] * 4
        + [row_spec] * 4
        + [row_spec, pl.BlockSpec((1, 1), lambda i: (0, 0))],
        out_specs=pl.BlockSpec((1, 1, M2), lambda i: (i, 0, 0)),
        out_shape=jax.ShapeDtypeStruct((N, 1, M2), jnp.float32),
        scratch_shapes=[pltpu.VMEM((1, M2), jnp.float32)],
        interpret=_INTERPRET,
    )(x1c, y1c, x2c, y2c, x1r, y1r, x2r, y2r, valid_s, t_arr)

    keptb = kept[:, 0, :] > jnp.float32(0.5)
    seli = (keptb & (sc_s >= box_score_thresh)).astype(jnp.int32)
    pos = jnp.cumsum(seli, axis=-1) - 1
    cnt = jnp.sum(seli, axis=-1, keepdims=True)
    rank_ns = jnp.cumsum(1 - seli, axis=-1) - 1
    tgt_in = jnp.where(seli == 1, pos, cnt + rank_ns)
    base = (jnp.arange(N) * M)[:, None]
    dump = jnp.int32(N * M)
    tgt_g = jnp.where(tgt_in < M, tgt_in + base, dump).astype(jnp.int32)
    idx_sc = tgt_g.reshape(NW, _SC_KR, 128)

    packed = jnp.where((seli == 1)[..., None], packed_s, 0).reshape(N * M2, 8)

    out_flat = _sc_scatter(packed, idx_sc)
    body = out_flat[: N * M]
    out_boxes = lax.bitcast_convert_type(body[:, :4], jnp.float32).reshape(N, M, 4)
    out_scores = lax.bitcast_convert_type(body[:, 4], jnp.float32).reshape(N, M)
    out_labels = body[:, 5].astype(label_dtype).reshape(N, M)
    counts = cnt[:, 0].astype(jnp.int32)
    return (out_boxes, out_scores, out_labels, counts)
